# baseline (speedup 1.0000x reference)
import functools

import jax
import jax.numpy as jnp
from jax import lax
from jax.experimental import pallas as pl
from jax.experimental.pallas import tpu as pltpu
from jax.experimental.pallas import tpu_sc as plsc

K_STEPS = 10
ALPHA = 0.1
HID = 64
NC = 2
NS = 16
NW = NC * NS
CHUNK = 128
NBUF = 4


def _make_sc_scatter(n_pad, e_pad):
  epw = e_pad // NW
  rpt = n_pad // NS
  n_chunks = epw // CHUNK
  nbuf = NBUF
  assert n_chunks % nbuf == 0 and n_chunks // nbuf >= 2
  n_groups = n_chunks // nbuf
  mesh = plsc.VectorSubcoreMesh(core_axis_name="c", subcore_axis_name="s")

  @functools.partial(
      pl.kernel,
      out_type=jax.ShapeDtypeStruct((NC, n_pad, HID), jnp.float32),
      mesh=mesh,
      compiler_params=pltpu.CompilerParams(use_tc_tiling_on_sc=False),
      scratch_types=[
          pltpu.VMEM((n_chunks, CHUNK), jnp.int32),
          pltpu.VMEM((n_chunks, CHUNK), jnp.int32),
          [pltpu.VMEM((CHUNK, HID), jnp.float32) for _ in range(nbuf)],
          [pltpu.SemaphoreType.DMA for _ in range(nbuf)],
          [pltpu.SemaphoreType.DMA for _ in range(nbuf)],
          pltpu.VMEM_SHARED((n_pad, HID), jnp.float32),
      ],
  )
  def sc_scatter(table_hbm, row_hbm, col_hbm, out_hbm,
                 row_v, col_v, bufs, sems, ssems, acc_sh):
    c = lax.axis_index("c")
    s = lax.axis_index("s")
    wid = c * NS + s

    pltpu.sync_copy(row_hbm.at[wid], row_v)
    pltpu.sync_copy(col_hbm.at[wid], col_v)

    zeros16 = jnp.zeros((16,), jnp.float32)
    def zrow(i, carry):
      for j in range(HID // 16):
        bufs[0][i, pl.ds(j * 16, 16)] = zeros16
      return carry
    lax.fori_loop(0, CHUNK, zrow, 0)
    for z in range(rpt // CHUNK):
      pltpu.sync_copy(bufs[0], acc_sh.at[pl.ds(s * rpt + z * CHUNK, CHUNK)])
    plsc.subcore_barrier()

    def gather(j, b):
      pltpu.async_copy(table_hbm.at[row_v.at[j]], bufs[b], sems[b])

    def gwait(b):
      pltpu.make_async_copy(table_hbm.at[pl.ds(0, CHUNK)], bufs[b],
                            sems[b]).wait()

    def scatter(j, b):
      pltpu.async_copy(bufs[b], acc_sh.at[col_v.at[j]], ssems[b], add=True)

    def swait(b):
      pltpu.make_async_copy(table_hbm.at[pl.ds(0, CHUNK)], bufs[b],
                            ssems[b]).wait()

    for b in range(nbuf):
      gather(b, b)

    def group(g, carry):
      base = g * nbuf
      for b in range(nbuf):
        gwait(b)
        scatter(base + b, b)
      for b in range(nbuf):
        swait(b)
        gather(base + b + nbuf, b)
      return carry
    lax.fori_loop(0, n_groups - 1, group, 0)
    for b in range(nbuf):
      gwait(b)
      scatter((n_groups - 1) * nbuf + b, b)
    for b in range(nbuf):
      swait(b)

    plsc.subcore_barrier()
    pltpu.sync_copy(acc_sh.at[pl.ds(s * rpt, rpt)],
                    out_hbm.at[c, pl.ds(s * rpt, rpt)])

  return sc_scatter


def _tc_prep(x_pad, w1, b1, deg2):
  n_pad = x_pad.shape[0]
  br = 1024

  def body(x_ref, w1_ref, b1_ref, d2_ref, hs0_ref, a_ref, rec_ref):
    hp = lax.dot_general(x_ref[...], w1_ref[...],
                         (((1,), (1,)), ((), ())),
                         preferred_element_type=jnp.float32) + b1_ref[...]
    deg = d2_ref[0] + d2_ref[1] + 1.0
    dinv = lax.rsqrt(deg)
    hs0_ref[...] = dinv * hp
    a_ref[...] = (1.0 - ALPHA) / deg
    rec_ref[...] = jnp.sqrt(deg)

  grid = (n_pad // br,)
  return pl.pallas_call(
      body,
      grid=grid,
      in_specs=[
          pl.BlockSpec((br, x_pad.shape[1]), lambda i: (i, 0)),
          pl.BlockSpec((HID, x_pad.shape[1]), lambda i: (0, 0)),
          pl.BlockSpec((1, HID), lambda i: (0, 0)),
          pl.BlockSpec((NC, br, HID), lambda i: (0, i, 0)),
      ],
      out_specs=[
          pl.BlockSpec((br, HID), lambda i: (i, 0)),
          pl.BlockSpec((br, HID), lambda i: (i, 0)),
          pl.BlockSpec((br, HID), lambda i: (i, 0)),
      ],
      out_shape=[jax.ShapeDtypeStruct((n_pad, HID), jnp.float32)] * 3,
  )(x_pad, w1, b1, deg2)


def _tc_update(s2, hs, a, hs0):
  def body(s2_ref, hs_ref, a_ref, hs0_ref, out_ref):
    out_ref[...] = (a_ref[...] * (s2_ref[0] + s2_ref[1] + hs_ref[...])
                    + ALPHA * hs0_ref[...])

  return pl.pallas_call(
      body,
      out_shape=jax.ShapeDtypeStruct(hs.shape, jnp.float32),
  )(s2, hs, a, hs0)


def _tc_out(hs, rec, w2, b2):
  def body(hs_ref, rec_ref, w2_ref, b2_ref, out_ref):
    h = rec_ref[...] * hs_ref[...]
    out_ref[...] = lax.dot_general(h, w2_ref[...], (((1,), (1,)), ((), ())),
                                   preferred_element_type=jnp.float32) + b2_ref[...]

  return pl.pallas_call(
      body,
      out_shape=jax.ShapeDtypeStruct((hs.shape[0], w2.shape[0]), jnp.float32),
  )(hs, rec, w2, b2)


def kernel(x, edge_index, W1, b1, W2, b2):
  n = x.shape[0]
  e = edge_index.shape[1]
  n_pad = ((n + NS * 8 - 1) // (NS * 8)) * (NS * 8) + NS * 8
  egrain = NW * CHUNK * NBUF
  e_pad = ((e + egrain - 1) // egrain) * egrain

  row = edge_index[0].astype(jnp.int32)
  col = edge_index[1].astype(jnp.int32)
  order = jnp.argsort(col)
  row = row[order]
  col = col[order]
  pad = e_pad - e
  rowp = jnp.concatenate([row, jnp.zeros((pad,), jnp.int32)]).reshape(NW, -1, CHUNK)
  colp = jnp.concatenate([col, jnp.full((pad,), n_pad - 1, jnp.int32)]).reshape(NW, -1, CHUNK)

  sc_scatter = _make_sc_scatter(n_pad, e_pad)

  ones_tab = jnp.ones((n_pad, HID), jnp.float32)
  deg2 = sc_scatter(ones_tab, rowp, colp)

  x_pad = jnp.pad(x, ((0, n_pad - n), (0, 0)))
  hs0, a, rec = _tc_prep(x_pad, W1, b1.reshape(1, HID), deg2)

  hs = hs0
  for _ in range(K_STEPS):
    s2 = sc_scatter(hs, rowp, colp)
    hs = _tc_update(s2, hs, a, hs0)

  logits = _tc_out(hs, rec, W2, b2.reshape(1, -1))
  return logits[:n]


# speedup vs baseline: 1.2033x; 1.2033x over previous
import functools

import jax
import jax.numpy as jnp
from jax import lax
from jax.experimental import pallas as pl
from jax.experimental.pallas import tpu as pltpu
from jax.experimental.pallas import tpu_sc as plsc

K_STEPS = 10
ALPHA = 0.1
HID = 64
NC = 2
NS = 16
F = HID // NS
CHUNKE = 4096
NBUF = 2


def _make_sc_segsum(n_pad, e_pad):
  n_ch = e_pad // (NC * CHUNKE)
  n_pairs = n_ch // NBUF
  assert n_ch % NBUF == 0 and n_pairs >= 2
  n_groups = CHUNKE // 16
  mesh = plsc.VectorSubcoreMesh(core_axis_name="c", subcore_axis_name="s")

  @functools.partial(
      pl.kernel,
      out_type=jax.ShapeDtypeStruct((NC, HID, n_pad), jnp.float32),
      mesh=mesh,
      compiler_params=pltpu.CompilerParams(use_tc_tiling_on_sc=False,
                                           needs_layout_passes=False),
      scratch_types=[
          pltpu.VMEM((F, n_pad), jnp.float32),
          pltpu.VMEM((F, n_pad), jnp.float32),
          [pltpu.VMEM((CHUNKE,), jnp.int32) for _ in range(NBUF)],
          [pltpu.VMEM((CHUNKE,), jnp.int32) for _ in range(NBUF)],
          [pltpu.SemaphoreType.DMA for _ in range(NBUF)],
          [pltpu.SemaphoreType.DMA for _ in range(NBUF)],
      ],
  )
  def sc_segsum(hsT_hbm, row_hbm, col_hbm, out_hbm,
                table_v, acc_v, rbufs, cbufs, rsems, csems):
    c = lax.axis_index("c")
    s = lax.axis_index("s")

    pltpu.sync_copy(hsT_hbm.at[pl.ds(s * F, F)], table_v)

    zeros16 = jnp.zeros((16,), jnp.float32)
    def zrow(i, carry):
      for f in range(F):
        acc_v[f, pl.ds(i * 16, 16)] = zeros16
      return carry
    lax.fori_loop(0, n_pad // 16, zrow, 0)

    def issue(j, b):
      pltpu.async_copy(row_hbm.at[c, j], rbufs[b], rsems[b])
      pltpu.async_copy(col_hbm.at[c, j], cbufs[b], csems[b])

    def wait(b):
      pltpu.make_async_copy(row_hbm.at[c, 0], rbufs[b], rsems[b]).wait()
      pltpu.make_async_copy(col_hbm.at[c, 0], cbufs[b], csems[b]).wait()

    fidx = [jnp.full((16,), f, jnp.int32) for f in range(F)]

    def process(b):
      def group(g, dupv):
        rowv = rbufs[b][pl.ds(g * 16, 16)]
        colv = cbufs[b][pl.ds(g * 16, 16)]
        cnt, _ = plsc.scan_count(colv)
        vals = [plsc.load_gather(table_v, [fidx[f], rowv]) for f in range(F)]
        for k in (1, 2):
          mk = cnt == k
          for f in range(F):
            plsc.addupdate_scatter(acc_v, [fidx[f], colv], vals[f], mask=mk)
        return jnp.maximum(dupv, cnt)
      dupv = lax.fori_loop(0, n_groups, group, jnp.zeros((16,), jnp.int32))
      dmax = lax.reduce_max(dupv, (0,))

      @pl.when(dmax > 2)
      def _slow():
        def group2(g, carry):
          rowv = rbufs[b][pl.ds(g * 16, 16)]
          colv = cbufs[b][pl.ds(g * 16, 16)]
          cnt, _ = plsc.scan_count(colv)
          vals = [plsc.load_gather(table_v, [fidx[f], rowv])
                  for f in range(F)]
          for k in range(3, 17):
            mk = cnt == k
            for f in range(F):
              plsc.addupdate_scatter(acc_v, [fidx[f], colv], vals[f],
                                     mask=mk)
          return carry
        lax.fori_loop(0, n_groups, group2, 0)

    for b in range(NBUF):
      issue(b, b)

    def pair(j2, carry):
      for b in range(NBUF):
        wait(b)
        process(b)
        issue(j2 * NBUF + b + NBUF, b)
      return carry
    lax.fori_loop(0, n_pairs - 1, pair, 0)
    for b in range(NBUF):
      wait(b)
      process(b)

    pltpu.sync_copy(acc_v, out_hbm.at[c, pl.ds(s * F, F)])

  return sc_segsum


def _tc_prep(x_pad, w1, b1, deg2):
  n_pad = x_pad.shape[0]

  def body(x_ref, w1_ref, b1_ref, d2_ref, hs0_ref, a_ref, rec_ref):
    hpT = lax.dot_general(w1_ref[...], x_ref[...],
                          (((1,), (1,)), ((), ())),
                          preferred_element_type=jnp.float32) + b1_ref[...]
    degT = d2_ref[0] + d2_ref[1] + 1.0
    dinvT = lax.rsqrt(degT)
    hs0_ref[...] = dinvT * hpT
    a_ref[...] = (1.0 - ALPHA) / degT
    rec_ref[...] = jnp.sqrt(degT)

  return pl.pallas_call(
      body,
      out_shape=[jax.ShapeDtypeStruct((HID, n_pad), jnp.float32)] * 3,
  )(x_pad, w1, b1, deg2)


def _tc_update(s2, hs, a, hs0):
  def body(s2_ref, hs_ref, a_ref, hs0_ref, out_ref):
    out_ref[...] = (a_ref[...] * (s2_ref[0] + s2_ref[1] + hs_ref[...])
                    + ALPHA * hs0_ref[...])

  return pl.pallas_call(
      body,
      out_shape=jax.ShapeDtypeStruct(hs.shape, jnp.float32),
  )(s2, hs, a, hs0)


def _tc_out(hs, rec, w2, b2):
  def body(hs_ref, rec_ref, w2_ref, b2_ref, out_ref):
    h = rec_ref[...] * hs_ref[...]
    out_ref[...] = lax.dot_general(h, w2_ref[...], (((0,), (1,)), ((), ())),
                                   preferred_element_type=jnp.float32) + b2_ref[...]

  return pl.pallas_call(
      body,
      out_shape=jax.ShapeDtypeStruct((hs.shape[1], w2.shape[0]), jnp.float32),
  )(hs, rec, w2, b2)


def kernel(x, edge_index, W1, b1, W2, b2):
  n = x.shape[0]
  e = edge_index.shape[1]
  n_pad = ((n + 16 + 127) // 128) * 128 + 128
  egrain = NC * CHUNKE * NBUF
  e_pad = ((e + egrain - 1) // egrain) * egrain

  row = edge_index[0].astype(jnp.int32)
  col = edge_index[1].astype(jnp.int32)
  pad = e_pad - e
  sink = n_pad - 16 + (jnp.arange(pad, dtype=jnp.int32) % 16)
  rowp = jnp.concatenate([row, jnp.zeros((pad,), jnp.int32)])
  colp = jnp.concatenate([col, sink])
  rowp = rowp.reshape(NC, -1, CHUNKE)
  colp = colp.reshape(NC, -1, CHUNKE)

  sc_segsum = _make_sc_segsum(n_pad, e_pad)

  onesT = jnp.ones((HID, n_pad), jnp.float32)
  deg2 = sc_segsum(onesT, rowp, colp)

  x_pad = jnp.pad(x, ((0, n_pad - n), (0, 0)))
  hs0, a, rec = _tc_prep(x_pad, W1, b1.reshape(HID, 1), deg2)

  hs = hs0
  for _ in range(K_STEPS):
    s2 = sc_segsum(hs, rowp, colp)
    hs = _tc_update(s2, hs, a, hs0)

  logits = _tc_out(hs, rec, W2, b2.reshape(1, -1))
  return logits[:n]
